# baseline (speedup 1.0000x reference)
import functools

import jax
import jax.numpy as jnp
from jax import lax
from jax.experimental import pallas as pl
from jax.experimental.pallas import tpu as pltpu
from jax.experimental.pallas import tpu_sc as plsc

N = 10000
E = 160000
IN_DIM = 256
HID = 512
MASK_RATE = 0.3
NUM_MASK = int(MASK_RATE * N)

NC = 2
NS = 16
EPT = E // NS
BLK = 32
PLEN = 10176
ACC_ROWS = 2512
NP = 10112
ECH = 2000

_MASK_NP = None


def _mask_const():
    global _MASK_NP
    if _MASK_NP is None:
        perm = jax.random.permutation(jax.random.key(42), N)
        m = jnp.zeros((N,), jnp.float32).at[perm[:NUM_MASK]].set(1.0)
        _MASK_NP = jax.device_get(m)
    return jnp.asarray(_MASK_NP).reshape(N, 1)


def _sc_mesh():
    return plsc.VectorSubcoreMesh(core_axis_name="c", subcore_axis_name="s")


def _deg_body(srch, dsth, dego, degi, eidx, hist, tbuf, abuf, sh2):
    c0 = lax.axis_index("c")
    s = lax.axis_index("s")

    @pl.when(c0 == 0)
    def _():
        pltpu.sync_copy(srch.at[pl.ds(s * EPT, EPT)], eidx)

    @pl.when(c0 == 1)
    def _():
        pltpu.sync_copy(dsth.at[pl.ds(s * EPT, EPT)], eidx)
    zero16 = jnp.zeros((16,), jnp.float32)

    def zloop(i, _):
        hist[pl.ds(i * 16, 16)] = zero16
        return 0

    lax.fori_loop(0, NP // 16, zloop, 0)

    ones16 = jnp.ones((16,), jnp.float32)

    def acc(i, _):
        idx = eidx[pl.ds(i * 16, 16)]
        plsc.addupdate_scatter(hist, [idx], ones16)
        return 0

    lax.fori_loop(0, EPT // 16, acc, 0)

    pltpu.sync_copy(hist, sh2.at[s])
    plsc.subcore_barrier()

    @pl.when(s < NS - 1)
    def _():
        _deg_reduce(c0, s * 640, 640, dego, degi, tbuf, abuf, sh2)

    @pl.when(s == NS - 1)
    def _():
        _deg_reduce(c0, (NS - 1) * 640, NP - (NS - 1) * 640, dego, degi,
                    tbuf, abuf, sh2)


def _deg_reduce(c0, off, ln, dego, degi, tbuf, abuf, sh2):
    zero16 = jnp.zeros((16,), jnp.float32)

    def z(i, _):
        abuf[pl.ds(i * 16, 16)] = zero16
        return 0

    lax.fori_loop(0, ln // 16, z, 0)
    for t in range(NS):
        pltpu.sync_copy(sh2.at[t, pl.ds(off, ln)], tbuf.at[pl.ds(0, ln)])

        def ad(i, _):
            abuf[pl.ds(i * 16, 16)] = (abuf[pl.ds(i * 16, 16)]
                                       + tbuf[pl.ds(i * 16, 16)])
            return 0

        lax.fori_loop(0, ln // 16, ad, 0)

    @pl.when(c0 == 0)
    def _():
        pltpu.sync_copy(abuf.at[pl.ds(0, ln)], dego.at[pl.ds(off, ln)])

    @pl.when(c0 == 1)
    def _():
        pltpu.sync_copy(abuf.at[pl.ds(0, ln)], degi.at[pl.ds(off, ln)])


def _deg_call(edge_index):
    kf = pl.kernel(
        _deg_body,
        out_type=[
            jax.ShapeDtypeStruct((NP,), jnp.float32),
            jax.ShapeDtypeStruct((NP,), jnp.float32),
        ],
        mesh=_sc_mesh(),
        scratch_types=[
            pltpu.VMEM((EPT,), jnp.int32),
            pltpu.VMEM((NP,), jnp.float32),
            pltpu.VMEM((640,), jnp.float32),
            pltpu.VMEM((640,), jnp.float32),
            pltpu.VMEM_SHARED((NS, NP), jnp.float32),
        ],
        compiler_params=pltpu.CompilerParams(needs_layout_passes=False),
    )
    return kf(edge_index[0], edge_index[1])


QB = (0, 2504, 5008, 7512)
QS = (2504, 2504, 2504, 2488)


def _agg_pass(h, out, acc, partX, dst2d, gs0, gs1, rows0, rows1,
              sem0, sem1, s, smul, chunk, qbase, qsize, roff, start, nb):
    qdum = jnp.full((16,), qsize, jnp.int32)

    zerof = jnp.zeros((16,), jnp.float32)

    def zr(i, _):
        def zcol(j, _):
            rows0[i, 0, pl.ds(j * 16, 16)] = zerof
            rows0[i, 1, pl.ds(j * 16, 16)] = zerof
            return 0
        return lax.fori_loop(0, 128 // 16, zcol, 0)

    lax.fori_loop(0, 8, zr, 0)
    zbase = s * 160

    def zc(q, _):
        pltpu.sync_copy(rows0.at[pl.ds(0, 8)],
                        acc.at[pl.ds(zbase + q * 8, 8)])
        return 0

    @pl.when(s < NS - 1)
    def _():
        lax.fori_loop(0, 20, zc, 0)

    @pl.when(s == NS - 1)
    def _():
        lax.fori_loop(0, (ACC_ROWS - 2400) // 8, zc, 0)

    def bi(b, _):
        for j in range(BLK // 16):
            v = partX[pl.ds(start + b * BLK + j * 16, 16)]
            d = (v >> 16) - qbase
            dst2d[b, pl.ds(j * 16, 16)] = jnp.minimum(d, qdum)
        return 0

    lax.fori_loop(0, nb, bi, 0)
    plsc.subcore_barrier()

    def prep(b, gsbuf):
        for j in range(BLK // 16):
            v = partX[pl.ds(start + b * BLK + j * 16, 16)]
            gsbuf[pl.ds(j * 16, 16)] = (v & 0xFFFF) * smul + chunk

    def gat(gsbuf, buf, gsm):
        pltpu.async_copy(h.at[gsbuf], buf, gsm)

    @pl.when(nb >= 1)
    def _():
        prep(0, gs0)
        gat(gs0, rows0, sem0)

    def pair(p, _):
        b0 = p * 2
        prep(b0 + 1, gs1)
        gat(gs1, rows1, sem1)
        pltpu.make_async_copy(h.at[gs0], rows0, sem0).wait()
        pltpu.sync_copy(rows0, acc.at[dst2d.at[b0]], add=True)

        @pl.when(b0 + 2 < nb)
        def _():
            prep(b0 + 2, gs0)
            gat(gs0, rows0, sem0)

        pltpu.make_async_copy(h.at[gs1], rows1, sem1).wait()
        pltpu.sync_copy(rows1, acc.at[dst2d.at[b0 + 1]], add=True)
        return 0

    lax.fori_loop(0, nb // 2, pair, 0)

    @pl.when(nb % 2 == 1)
    def _():
        pltpu.make_async_copy(h.at[gs0], rows0, sem0).wait()
        pltpu.sync_copy(rows0, acc.at[dst2d.at[nb - 1]], add=True)

    plsc.subcore_barrier()

    @pl.when(s < NS - 1)
    def _():
        pltpu.sync_copy(acc.at[pl.ds(s * 160, 160)],
                        out.at[pl.ds(roff + s * 160, 160)])

    @pl.when(s == NS - 1)
    def _():
        pltpu.sync_copy(acc.at[pl.ds(2400, qsize - 2400)],
                        out.at[pl.ds(roff + 2400, qsize - 2400)])

    plsc.subcore_barrier()


def _agg_body(ct, h, srch, dsth, out, rawS, rawD, partP1, partP2, dst2d,
              gs0, gs1, rows0, rows1, acc, sem0, sem1):
    c0 = lax.axis_index("c")
    s = lax.axis_index("s")
    base = s * EPT

    padv = jnp.full((16,), (10000 << 16), jnp.int32)

    def pf(i, _):
        partP1[pl.ds(i * 16, 16)] = padv
        partP2[pl.ds(i * 16, 16)] = padv
        return 0

    lax.fori_loop(0, (PLEN + 16) // 16, pf, 0)

    def part_chunk(ci, state):
        a1, b1, a2, b2 = state
        pltpu.sync_copy(srch.at[pl.ds(base + ci * ECH, ECH)], rawS)
        pltpu.sync_copy(dsth.at[pl.ds(base + ci * ECH, ECH)], rawD)

        def grp(g, st):
            oa1, ob1, oa2, ob2 = st
            vs = rawS[pl.ds(g * 16, 16)]
            vd = rawD[pl.ds(g * 16, 16)]
            packed = (vd << 16) | vs
            l0 = vd < QB[1]
            l1 = vd < QB[2]
            l2 = vd < QB[3]
            mq0 = l0
            mq1 = jnp.logical_and(l1, jnp.logical_not(l0))
            mq2 = jnp.logical_and(l2, jnp.logical_not(l1))
            mq3 = jnp.logical_not(l2)
            pc0 = plsc.all_reduce_population_count(mq0)[0]
            pc1 = plsc.all_reduce_population_count(mq1)[0]
            pc2 = plsc.all_reduce_population_count(mq2)[0]
            pc3 = 16 - pc0 - pc1 - pc2
            plsc.store_compressed(partP1.at[pl.ds(oa1, 16)], packed, mask=mq0)
            nb1 = ob1 - pc1
            plsc.store_compressed(partP1.at[pl.ds(nb1, 16)], packed, mask=mq1)
            plsc.store_compressed(partP2.at[pl.ds(oa2, 16)], packed, mask=mq2)
            nb2 = ob2 - pc3
            plsc.store_compressed(partP2.at[pl.ds(nb2, 16)], packed, mask=mq3)
            return (oa1 + pc0, nb1, oa2 + pc2, nb2)

        return lax.fori_loop(0, ECH // 16, grp, (a1, b1, a2, b2))

    p0 = jnp.int32(0)
    pl_ = jnp.int32(PLEN)
    a1, b1, a2, b2 = lax.fori_loop(0, EPT // ECH, part_chunk,
                                   (p0, pl_, p0, pl_))
    nbQ0 = (a1 + BLK - 1) // BLK
    nbQ1 = (PLEN - b1 + BLK - 1) // BLK
    nbQ2 = (a2 + BLK - 1) // BLK
    nbQ3 = (PLEN - b2 + BLK - 1) // BLK
    st1 = PLEN - nbQ1 * BLK
    st3 = PLEN - nbQ3 * BLK

    plan = [
        (partP1, 0, nbQ0, 0),
        (partP1, st1, nbQ1, 1),
        (partP2, 0, nbQ2, 2),
        (partP2, st3, nbQ3, 3),
    ]

    if ct == 2:
        for partX, start, nb, q in plan:
            _agg_pass(h, out, acc, partX, dst2d, gs0, gs1, rows0, rows1,
                      sem0, sem1, s, 2, c0, QB[q], QS[q], c0 * N + QB[q],
                      start, nb)
    else:
        @pl.when(c0 == 0)
        def _():
            for partX, start, nb, q in plan[:2]:
                _agg_pass(h, out, acc, partX, dst2d, gs0, gs1, rows0, rows1,
                          sem0, sem1, s, 1, 0, QB[q], QS[q], QB[q], start, nb)

        @pl.when(c0 == 1)
        def _():
            for partX, start, nb, q in plan[2:]:
                _agg_pass(h, out, acc, partX, dst2d, gs0, gs1, rows0, rows1,
                          sem0, sem1, s, 1, 0, QB[q], QS[q], QB[q], start, nb)


def _agg_call(h_flat, src, dst, ct):
    kf = pl.kernel(
        functools.partial(_agg_body, ct),
        out_type=jax.ShapeDtypeStruct((ct * N, 2, 128), jnp.float32),
        mesh=_sc_mesh(),
        scratch_types=[
            pltpu.VMEM((ECH,), jnp.int32),
            pltpu.VMEM((ECH,), jnp.int32),
            pltpu.VMEM((PLEN + 16,), jnp.int32),
            pltpu.VMEM((PLEN + 16,), jnp.int32),
            pltpu.VMEM((PLEN // BLK, BLK), jnp.int32),
            pltpu.VMEM((BLK,), jnp.int32),
            pltpu.VMEM((BLK,), jnp.int32),
            pltpu.VMEM((BLK, 2, 128), jnp.float32),
            pltpu.VMEM((BLK, 2, 128), jnp.float32),
            pltpu.VMEM_SHARED((ACC_ROWS, 2, 128), jnp.float32),
            pltpu.SemaphoreType.DMA,
            pltpu.SemaphoreType.DMA,
        ],
        compiler_params=pltpu.CompilerParams(needs_layout_passes=False),
    )
    return kf(h_flat, src, dst)


RB = 1000
_DOT = dict(preferred_element_type=jnp.float32, precision=lax.Precision.HIGHEST)


def _enc1_body(xb, mb, tokb, w1b, dob, ob):
    xs = jnp.where(mb[...] > 0.0, tokb[...], xb[...])
    ns = lax.rsqrt(jnp.maximum(dob[...], 1.0))
    ob[...] = jnp.dot(xs, w1b[...], **_DOT) * ns


def _enc1_call(x, mask, tok, W1, dego2):
    return pl.pallas_call(
        _enc1_body,
        grid=(N // RB,),
        in_specs=[
            pl.BlockSpec((RB, IN_DIM), lambda r: (r, 0)),
            pl.BlockSpec((RB, 1), lambda r: (r, 0)),
            pl.BlockSpec((1, IN_DIM), lambda r: (0, 0)),
            pl.BlockSpec((IN_DIM, HID), lambda r: (0, 0)),
            pl.BlockSpec((RB, 1), lambda r: (r, 0)),
        ],
        out_specs=pl.BlockSpec((RB, HID), lambda r: (r, 0)),
        out_shape=jax.ShapeDtypeStruct((N, HID), jnp.float32),
    )(x, mask, tok, W1, dego2)


def _relu_cat(ab, ndb, bb):
    parts = [
        jnp.maximum(ab[c] * ndb + bb[:, c * 256:(c + 1) * 256], 0.0)
        for c in range(2)
    ]
    return jnp.concatenate(parts, axis=-1)


def _enc2_body(ab, dib, dob, b1b, w2b, ob):
    nd = lax.rsqrt(jnp.maximum(dib[...], 1.0))
    ns = lax.rsqrt(jnp.maximum(dob[...], 1.0))
    h1 = _relu_cat(ab, nd, b1b)
    ob[...] = jnp.dot(h1, w2b[...], **_DOT) * ns


def _enc2_call(agg1, degi2, dego2, b1, W2):
    return pl.pallas_call(
        _enc2_body,
        grid=(N // RB,),
        in_specs=[
            pl.BlockSpec((2, RB, 256), lambda r: (0, r, 0)),
            pl.BlockSpec((RB, 1), lambda r: (r, 0)),
            pl.BlockSpec((RB, 1), lambda r: (r, 0)),
            pl.BlockSpec((1, HID), lambda r: (0, 0)),
            pl.BlockSpec((HID, HID), lambda r: (0, 0)),
        ],
        out_specs=pl.BlockSpec((RB, HID), lambda r: (r, 0)),
        out_shape=jax.ShapeDtypeStruct((N, HID), jnp.float32),
    )(agg1, degi2, dego2, b1, W2)


def _wde_body(ab, bb, ob):
    ob[...] = jnp.dot(ab[...], bb[...], **_DOT)


def _wde_call(We2d, Wd):
    return pl.pallas_call(
        _wde_body,
        out_shape=jax.ShapeDtypeStruct((HID, IN_DIM), jnp.float32),
    )(We2d, Wd)


def _l2n(p):
    return p * lax.rsqrt(jnp.maximum(jnp.sum(p * p, axis=-1, keepdims=True),
                                     1e-24))


def _heads_body(ab, dib, dob, b2b, wmb, bmb, wvb, bvb, wdeb, maskb,
                mean_o, var_o, d_o):
    nd = lax.rsqrt(jnp.maximum(dib[...], 1.0))
    ns = lax.rsqrt(jnp.maximum(dob[...], 1.0))
    h2 = _relu_cat(ab, nd, b2b)
    mp = jnp.dot(h2, wmb[...], **_DOT) + bmb[...]
    vp = jnp.dot(h2, wvb[...], **_DOT) + bvb[...]
    mean_o[...] = _l2n(mp)
    var_o[...] = _l2n(vp)
    d_o[...] = jnp.dot(h2, wdeb[...], **_DOT) * ns * (1.0 - maskb[...])


def _heads_call(agg2, degi2, dego2, b2, Wm, bm, Wv, bv, wde, mask):
    return pl.pallas_call(
        _heads_body,
        grid=(N // RB,),
        in_specs=[
            pl.BlockSpec((2, RB, 256), lambda r: (0, r, 0)),
            pl.BlockSpec((RB, 1), lambda r: (r, 0)),
            pl.BlockSpec((RB, 1), lambda r: (r, 0)),
            pl.BlockSpec((1, HID), lambda r: (0, 0)),
            pl.BlockSpec((HID, HID), lambda r: (0, 0)),
            pl.BlockSpec((1, HID), lambda r: (0, 0)),
            pl.BlockSpec((HID, HID), lambda r: (0, 0)),
            pl.BlockSpec((1, HID), lambda r: (0, 0)),
            pl.BlockSpec((HID, IN_DIM), lambda r: (0, 0)),
            pl.BlockSpec((RB, 1), lambda r: (r, 0)),
        ],
        out_specs=[
            pl.BlockSpec((RB, HID), lambda r: (r, 0)),
            pl.BlockSpec((RB, HID), lambda r: (r, 0)),
            pl.BlockSpec((RB, IN_DIM), lambda r: (r, 0)),
        ],
        out_shape=[
            jax.ShapeDtypeStruct((N, HID), jnp.float32),
            jax.ShapeDtypeStruct((N, HID), jnp.float32),
            jax.ShapeDtypeStruct((N, IN_DIM), jnp.float32),
        ],
    )(agg2, degi2, dego2, b2, Wm, bm, Wv, bv, wde, mask)


def _dec_body(ab, dib, bdb, ob):
    nd = lax.rsqrt(jnp.maximum(dib[...], 1.0))
    ob[...] = ab[...] * nd + bdb[...]


def _dec_call(agg3, degi2, bd):
    return pl.pallas_call(
        _dec_body,
        grid=(N // RB,),
        in_specs=[
            pl.BlockSpec((RB, IN_DIM), lambda r: (r, 0)),
            pl.BlockSpec((RB, 1), lambda r: (r, 0)),
            pl.BlockSpec((1, IN_DIM), lambda r: (0, 0)),
        ],
        out_specs=pl.BlockSpec((RB, IN_DIM), lambda r: (r, 0)),
        out_shape=jax.ShapeDtypeStruct((N, IN_DIM), jnp.float32),
    )(agg3, degi2, bd)


def kernel(x, edge_index, enc_mask_token, W1, b1, W2, b2, Wm, bm, Wv, bv,
           We2d, Wd, bd):
    mask = _mask_const()
    src = edge_index[0]
    dst = edge_index[1]
    dego, degi = _deg_call(edge_index)
    dego2 = dego[:N].reshape(N, 1)
    degi2 = degi[:N].reshape(N, 1)

    h1s = _enc1_call(x, mask, enc_mask_token, W1, dego2)
    agg1 = _agg_call(h1s.reshape(2 * N, 2, 128), src, dst, 2)
    h2s = _enc2_call(agg1.reshape(2, N, 256), degi2, dego2,
                     b1.reshape(1, HID), W2)
    agg2 = _agg_call(h2s.reshape(2 * N, 2, 128), src, dst, 2)
    wde = _wde_call(We2d, Wd)
    mean, var, dch = _heads_call(agg2.reshape(2, N, 256), degi2, dego2,
                                 b2.reshape(1, HID), Wm, bm.reshape(1, HID),
                                 Wv, bv.reshape(1, HID), wde, mask)
    agg3 = _agg_call(dch.reshape(N, 2, 128), src, dst, 1)
    recon = _dec_call(agg3.reshape(N, IN_DIM), degi2, bd.reshape(1, IN_DIM))
    return (recon, mean, var)


# speedup vs baseline: 1.0165x; 1.0165x over previous
import functools

import jax
import jax.numpy as jnp
from jax import lax
from jax.experimental import pallas as pl
from jax.experimental.pallas import tpu as pltpu
from jax.experimental.pallas import tpu_sc as plsc

N = 10000
E = 160000
IN_DIM = 256
HID = 512
MASK_RATE = 0.3
NUM_MASK = int(MASK_RATE * N)

NC = 2
NS = 16
EPT = E // NS
BLK = 48
PLEN = 10176
ACC_ROWS = 2512
NP = 10112
ECH = 2000

_MASK_NP = None


def _mask_const():
    global _MASK_NP
    if _MASK_NP is None:
        perm = jax.random.permutation(jax.random.key(42), N)
        m = jnp.zeros((N,), jnp.float32).at[perm[:NUM_MASK]].set(1.0)
        _MASK_NP = jax.device_get(m)
    return jnp.asarray(_MASK_NP).reshape(N, 1)


def _sc_mesh():
    return plsc.VectorSubcoreMesh(core_axis_name="c", subcore_axis_name="s")


def _deg_body(srch, dsth, dego, degi, eidx, hist, tbuf, abuf, sh2):
    c0 = lax.axis_index("c")
    s = lax.axis_index("s")

    @pl.when(c0 == 0)
    def _():
        pltpu.sync_copy(srch.at[pl.ds(s * EPT, EPT)], eidx)

    @pl.when(c0 == 1)
    def _():
        pltpu.sync_copy(dsth.at[pl.ds(s * EPT, EPT)], eidx)
    zero16 = jnp.zeros((16,), jnp.float32)

    def zloop(i, _):
        hist[pl.ds(i * 16, 16)] = zero16
        return 0

    lax.fori_loop(0, NP // 16, zloop, 0)

    ones16 = jnp.ones((16,), jnp.float32)

    def acc(i, _):
        idx = eidx[pl.ds(i * 16, 16)]
        plsc.addupdate_scatter(hist, [idx], ones16)
        return 0

    lax.fori_loop(0, EPT // 16, acc, 0)

    pltpu.sync_copy(hist, sh2.at[s])
    plsc.subcore_barrier()

    @pl.when(s < NS - 1)
    def _():
        _deg_reduce(c0, s * 640, 640, dego, degi, tbuf, abuf, sh2)

    @pl.when(s == NS - 1)
    def _():
        _deg_reduce(c0, (NS - 1) * 640, NP - (NS - 1) * 640, dego, degi,
                    tbuf, abuf, sh2)


def _deg_reduce(c0, off, ln, dego, degi, tbuf, abuf, sh2):
    zero16 = jnp.zeros((16,), jnp.float32)

    def z(i, _):
        abuf[pl.ds(i * 16, 16)] = zero16
        return 0

    lax.fori_loop(0, ln // 16, z, 0)
    for t in range(NS):
        pltpu.sync_copy(sh2.at[t, pl.ds(off, ln)], tbuf.at[pl.ds(0, ln)])

        def ad(i, _):
            abuf[pl.ds(i * 16, 16)] = (abuf[pl.ds(i * 16, 16)]
                                       + tbuf[pl.ds(i * 16, 16)])
            return 0

        lax.fori_loop(0, ln // 16, ad, 0)

    @pl.when(c0 == 0)
    def _():
        pltpu.sync_copy(abuf.at[pl.ds(0, ln)], dego.at[pl.ds(off, ln)])

    @pl.when(c0 == 1)
    def _():
        pltpu.sync_copy(abuf.at[pl.ds(0, ln)], degi.at[pl.ds(off, ln)])


def _deg_call(edge_index):
    kf = pl.kernel(
        _deg_body,
        out_type=[
            jax.ShapeDtypeStruct((NP,), jnp.float32),
            jax.ShapeDtypeStruct((NP,), jnp.float32),
        ],
        mesh=_sc_mesh(),
        scratch_types=[
            pltpu.VMEM((EPT,), jnp.int32),
            pltpu.VMEM((NP,), jnp.float32),
            pltpu.VMEM((640,), jnp.float32),
            pltpu.VMEM((640,), jnp.float32),
            pltpu.VMEM_SHARED((NS, NP), jnp.float32),
        ],
        compiler_params=pltpu.CompilerParams(needs_layout_passes=False),
    )
    return kf(edge_index[0], edge_index[1])


QB = (0, 2504, 5008, 7512)
QS = (2504, 2504, 2504, 2488)


def _agg_pass(h, out, acc, partX, dst2d, gs0, gs1, rows0, rows1,
              sem0, sem1, s, smul, chunk, qbase, qsize, roff, start, nb):
    qdum = jnp.full((16,), qsize, jnp.int32)

    zerof = jnp.zeros((16,), jnp.float32)

    def zr(i, _):
        def zcol(j, _):
            rows0[i, 0, pl.ds(j * 16, 16)] = zerof
            rows0[i, 1, pl.ds(j * 16, 16)] = zerof
            return 0
        return lax.fori_loop(0, 128 // 16, zcol, 0)

    lax.fori_loop(0, 8, zr, 0)
    zbase = s * 160

    def zc(q, _):
        pltpu.sync_copy(rows0.at[pl.ds(0, 8)],
                        acc.at[pl.ds(zbase + q * 8, 8)])
        return 0

    @pl.when(s < NS - 1)
    def _():
        lax.fori_loop(0, 20, zc, 0)

    @pl.when(s == NS - 1)
    def _():
        lax.fori_loop(0, (ACC_ROWS - 2400) // 8, zc, 0)

    def bi(b, _):
        for j in range(BLK // 16):
            v = partX[pl.ds(start + b * BLK + j * 16, 16)]
            d = (v >> 16) - qbase
            dst2d[b, pl.ds(j * 16, 16)] = jnp.minimum(d, qdum)
        return 0

    lax.fori_loop(0, nb, bi, 0)
    plsc.subcore_barrier()

    def prep(b, gsbuf):
        for j in range(BLK // 16):
            v = partX[pl.ds(start + b * BLK + j * 16, 16)]
            gsbuf[pl.ds(j * 16, 16)] = (v & 0xFFFF) * smul + chunk

    def gat(gsbuf, buf, gsm):
        pltpu.async_copy(h.at[gsbuf], buf, gsm)

    @pl.when(nb >= 1)
    def _():
        prep(0, gs0)
        gat(gs0, rows0, sem0)

    def pair(p, _):
        b0 = p * 2
        prep(b0 + 1, gs1)
        gat(gs1, rows1, sem1)
        pltpu.make_async_copy(h.at[gs0], rows0, sem0).wait()
        pltpu.sync_copy(rows0, acc.at[dst2d.at[b0]], add=True)

        @pl.when(b0 + 2 < nb)
        def _():
            prep(b0 + 2, gs0)
            gat(gs0, rows0, sem0)

        pltpu.make_async_copy(h.at[gs1], rows1, sem1).wait()
        pltpu.sync_copy(rows1, acc.at[dst2d.at[b0 + 1]], add=True)
        return 0

    lax.fori_loop(0, nb // 2, pair, 0)

    @pl.when(nb % 2 == 1)
    def _():
        pltpu.make_async_copy(h.at[gs0], rows0, sem0).wait()
        pltpu.sync_copy(rows0, acc.at[dst2d.at[nb - 1]], add=True)

    plsc.subcore_barrier()

    @pl.when(s < NS - 1)
    def _():
        pltpu.sync_copy(acc.at[pl.ds(s * 160, 160)],
                        out.at[pl.ds(roff + s * 160, 160)])

    @pl.when(s == NS - 1)
    def _():
        pltpu.sync_copy(acc.at[pl.ds(2400, qsize - 2400)],
                        out.at[pl.ds(roff + 2400, qsize - 2400)])

    plsc.subcore_barrier()


def _agg_body(ct, h, srch, dsth, out, rawS, rawD, partP1, partP2, dst2d,
              gs0, gs1, rows0, rows1, acc, sem0, sem1):
    c0 = lax.axis_index("c")
    s = lax.axis_index("s")
    base = s * EPT

    padv = jnp.full((16,), (10000 << 16), jnp.int32)

    def pf(i, _):
        partP1[pl.ds(i * 16, 16)] = padv
        partP2[pl.ds(i * 16, 16)] = padv
        return 0

    lax.fori_loop(0, (PLEN + 16) // 16, pf, 0)

    def part_chunk(ci, state):
        a1, b1, a2, b2 = state
        pltpu.sync_copy(srch.at[pl.ds(base + ci * ECH, ECH)], rawS)
        pltpu.sync_copy(dsth.at[pl.ds(base + ci * ECH, ECH)], rawD)

        def grp(g, st):
            oa1, ob1, oa2, ob2 = st
            vs = rawS[pl.ds(g * 16, 16)]
            vd = rawD[pl.ds(g * 16, 16)]
            packed = (vd << 16) | vs
            l0 = vd < QB[1]
            l1 = vd < QB[2]
            l2 = vd < QB[3]
            mq0 = l0
            mq1 = jnp.logical_and(l1, jnp.logical_not(l0))
            mq2 = jnp.logical_and(l2, jnp.logical_not(l1))
            mq3 = jnp.logical_not(l2)
            pc0 = plsc.all_reduce_population_count(mq0)[0]
            pc1 = plsc.all_reduce_population_count(mq1)[0]
            pc2 = plsc.all_reduce_population_count(mq2)[0]
            pc3 = 16 - pc0 - pc1 - pc2
            plsc.store_compressed(partP1.at[pl.ds(oa1, 16)], packed, mask=mq0)
            nb1 = ob1 - pc1
            plsc.store_compressed(partP1.at[pl.ds(nb1, 16)], packed, mask=mq1)
            plsc.store_compressed(partP2.at[pl.ds(oa2, 16)], packed, mask=mq2)
            nb2 = ob2 - pc3
            plsc.store_compressed(partP2.at[pl.ds(nb2, 16)], packed, mask=mq3)
            return (oa1 + pc0, nb1, oa2 + pc2, nb2)

        return lax.fori_loop(0, ECH // 16, grp, (a1, b1, a2, b2))

    p0 = jnp.int32(0)
    pl_ = jnp.int32(PLEN)
    a1, b1, a2, b2 = lax.fori_loop(0, EPT // ECH, part_chunk,
                                   (p0, pl_, p0, pl_))
    nbQ0 = (a1 + BLK - 1) // BLK
    nbQ1 = (PLEN - b1 + BLK - 1) // BLK
    nbQ2 = (a2 + BLK - 1) // BLK
    nbQ3 = (PLEN - b2 + BLK - 1) // BLK
    st1 = PLEN - nbQ1 * BLK
    st3 = PLEN - nbQ3 * BLK

    plan = [
        (partP1, 0, nbQ0, 0),
        (partP1, st1, nbQ1, 1),
        (partP2, 0, nbQ2, 2),
        (partP2, st3, nbQ3, 3),
    ]

    if ct == 2:
        for partX, start, nb, q in plan:
            _agg_pass(h, out, acc, partX, dst2d, gs0, gs1, rows0, rows1,
                      sem0, sem1, s, 2, c0, QB[q], QS[q], c0 * N + QB[q],
                      start, nb)
    else:
        @pl.when(c0 == 0)
        def _():
            for partX, start, nb, q in plan[:2]:
                _agg_pass(h, out, acc, partX, dst2d, gs0, gs1, rows0, rows1,
                          sem0, sem1, s, 1, 0, QB[q], QS[q], QB[q], start, nb)

        @pl.when(c0 == 1)
        def _():
            for partX, start, nb, q in plan[2:]:
                _agg_pass(h, out, acc, partX, dst2d, gs0, gs1, rows0, rows1,
                          sem0, sem1, s, 1, 0, QB[q], QS[q], QB[q], start, nb)


def _agg_call(h_flat, src, dst, ct):
    kf = pl.kernel(
        functools.partial(_agg_body, ct),
        out_type=jax.ShapeDtypeStruct((ct * N, 2, 128), jnp.float32),
        mesh=_sc_mesh(),
        scratch_types=[
            pltpu.VMEM((ECH,), jnp.int32),
            pltpu.VMEM((ECH,), jnp.int32),
            pltpu.VMEM((PLEN + 16,), jnp.int32),
            pltpu.VMEM((PLEN + 16,), jnp.int32),
            pltpu.VMEM((PLEN // BLK, BLK), jnp.int32),
            pltpu.VMEM((BLK,), jnp.int32),
            pltpu.VMEM((BLK,), jnp.int32),
            pltpu.VMEM((BLK, 2, 128), jnp.float32),
            pltpu.VMEM((BLK, 2, 128), jnp.float32),
            pltpu.VMEM_SHARED((ACC_ROWS, 2, 128), jnp.float32),
            pltpu.SemaphoreType.DMA,
            pltpu.SemaphoreType.DMA,
        ],
        compiler_params=pltpu.CompilerParams(needs_layout_passes=False),
    )
    return kf(h_flat, src, dst)


RB = 1000
_DOT = dict(preferred_element_type=jnp.float32, precision=lax.Precision.HIGHEST)


def _enc1_body(xb, mb, tokb, w1b, dob, ob):
    xs = jnp.where(mb[...] > 0.0, tokb[...], xb[...])
    ns = lax.rsqrt(jnp.maximum(dob[...], 1.0))
    ob[...] = jnp.dot(xs, w1b[...], **_DOT) * ns


def _enc1_call(x, mask, tok, W1, dego2):
    return pl.pallas_call(
        _enc1_body,
        grid=(N // RB,),
        in_specs=[
            pl.BlockSpec((RB, IN_DIM), lambda r: (r, 0)),
            pl.BlockSpec((RB, 1), lambda r: (r, 0)),
            pl.BlockSpec((1, IN_DIM), lambda r: (0, 0)),
            pl.BlockSpec((IN_DIM, HID), lambda r: (0, 0)),
            pl.BlockSpec((RB, 1), lambda r: (r, 0)),
        ],
        out_specs=pl.BlockSpec((RB, HID), lambda r: (r, 0)),
        out_shape=jax.ShapeDtypeStruct((N, HID), jnp.float32),
    )(x, mask, tok, W1, dego2)


def _relu_cat(ab, ndb, bb):
    parts = [
        jnp.maximum(ab[c] * ndb + bb[:, c * 256:(c + 1) * 256], 0.0)
        for c in range(2)
    ]
    return jnp.concatenate(parts, axis=-1)


def _enc2_body(ab, dib, dob, b1b, w2b, ob):
    nd = lax.rsqrt(jnp.maximum(dib[...], 1.0))
    ns = lax.rsqrt(jnp.maximum(dob[...], 1.0))
    h1 = _relu_cat(ab, nd, b1b)
    ob[...] = jnp.dot(h1, w2b[...], **_DOT) * ns


def _enc2_call(agg1, degi2, dego2, b1, W2):
    return pl.pallas_call(
        _enc2_body,
        grid=(N // RB,),
        in_specs=[
            pl.BlockSpec((2, RB, 256), lambda r: (0, r, 0)),
            pl.BlockSpec((RB, 1), lambda r: (r, 0)),
            pl.BlockSpec((RB, 1), lambda r: (r, 0)),
            pl.BlockSpec((1, HID), lambda r: (0, 0)),
            pl.BlockSpec((HID, HID), lambda r: (0, 0)),
        ],
        out_specs=pl.BlockSpec((RB, HID), lambda r: (r, 0)),
        out_shape=jax.ShapeDtypeStruct((N, HID), jnp.float32),
    )(agg1, degi2, dego2, b1, W2)


def _wde_body(ab, bb, ob):
    ob[...] = jnp.dot(ab[...], bb[...], **_DOT)


def _wde_call(We2d, Wd):
    return pl.pallas_call(
        _wde_body,
        out_shape=jax.ShapeDtypeStruct((HID, IN_DIM), jnp.float32),
    )(We2d, Wd)


def _l2n(p):
    return p * lax.rsqrt(jnp.maximum(jnp.sum(p * p, axis=-1, keepdims=True),
                                     1e-24))


def _heads_body(ab, dib, dob, b2b, wmb, bmb, wvb, bvb, wdeb, maskb,
                mean_o, var_o, d_o):
    nd = lax.rsqrt(jnp.maximum(dib[...], 1.0))
    ns = lax.rsqrt(jnp.maximum(dob[...], 1.0))
    h2 = _relu_cat(ab, nd, b2b)
    mp = jnp.dot(h2, wmb[...], **_DOT) + bmb[...]
    vp = jnp.dot(h2, wvb[...], **_DOT) + bvb[...]
    mean_o[...] = _l2n(mp)
    var_o[...] = _l2n(vp)
    d_o[...] = jnp.dot(h2, wdeb[...], **_DOT) * ns * (1.0 - maskb[...])


def _heads_call(agg2, degi2, dego2, b2, Wm, bm, Wv, bv, wde, mask):
    return pl.pallas_call(
        _heads_body,
        grid=(N // RB,),
        in_specs=[
            pl.BlockSpec((2, RB, 256), lambda r: (0, r, 0)),
            pl.BlockSpec((RB, 1), lambda r: (r, 0)),
            pl.BlockSpec((RB, 1), lambda r: (r, 0)),
            pl.BlockSpec((1, HID), lambda r: (0, 0)),
            pl.BlockSpec((HID, HID), lambda r: (0, 0)),
            pl.BlockSpec((1, HID), lambda r: (0, 0)),
            pl.BlockSpec((HID, HID), lambda r: (0, 0)),
            pl.BlockSpec((1, HID), lambda r: (0, 0)),
            pl.BlockSpec((HID, IN_DIM), lambda r: (0, 0)),
            pl.BlockSpec((RB, 1), lambda r: (r, 0)),
        ],
        out_specs=[
            pl.BlockSpec((RB, HID), lambda r: (r, 0)),
            pl.BlockSpec((RB, HID), lambda r: (r, 0)),
            pl.BlockSpec((RB, IN_DIM), lambda r: (r, 0)),
        ],
        out_shape=[
            jax.ShapeDtypeStruct((N, HID), jnp.float32),
            jax.ShapeDtypeStruct((N, HID), jnp.float32),
            jax.ShapeDtypeStruct((N, IN_DIM), jnp.float32),
        ],
    )(agg2, degi2, dego2, b2, Wm, bm, Wv, bv, wde, mask)


def _dec_body(ab, dib, bdb, ob):
    nd = lax.rsqrt(jnp.maximum(dib[...], 1.0))
    ob[...] = ab[...] * nd + bdb[...]


def _dec_call(agg3, degi2, bd):
    return pl.pallas_call(
        _dec_body,
        grid=(N // RB,),
        in_specs=[
            pl.BlockSpec((RB, IN_DIM), lambda r: (r, 0)),
            pl.BlockSpec((RB, 1), lambda r: (r, 0)),
            pl.BlockSpec((1, IN_DIM), lambda r: (0, 0)),
        ],
        out_specs=pl.BlockSpec((RB, IN_DIM), lambda r: (r, 0)),
        out_shape=jax.ShapeDtypeStruct((N, IN_DIM), jnp.float32),
    )(agg3, degi2, bd)


def kernel(x, edge_index, enc_mask_token, W1, b1, W2, b2, Wm, bm, Wv, bv,
           We2d, Wd, bd):
    mask = _mask_const()
    src = edge_index[0]
    dst = edge_index[1]
    dego, degi = _deg_call(edge_index)
    dego2 = dego[:N].reshape(N, 1)
    degi2 = degi[:N].reshape(N, 1)

    h1s = _enc1_call(x, mask, enc_mask_token, W1, dego2)
    agg1 = _agg_call(h1s.reshape(2 * N, 2, 128), src, dst, 2)
    h2s = _enc2_call(agg1.reshape(2, N, 256), degi2, dego2,
                     b1.reshape(1, HID), W2)
    agg2 = _agg_call(h2s.reshape(2 * N, 2, 128), src, dst, 2)
    wde = _wde_call(We2d, Wd)
    mean, var, dch = _heads_call(agg2.reshape(2, N, 256), degi2, dego2,
                                 b2.reshape(1, HID), Wm, bm.reshape(1, HID),
                                 Wv, bv.reshape(1, HID), wde, mask)
    agg3 = _agg_call(dch.reshape(N, 2, 128), src, dst, 1)
    recon = _dec_call(agg3.reshape(N, IN_DIM), degi2, bd.reshape(1, IN_DIM))
    return (recon, mean, var)
